# baseline (speedup 1.0000x reference)
import jax
import jax.numpy as jnp
from jax.experimental import pallas as pl
from jax.experimental.pallas import tpu as pltpu

_XR, _XC = 262144, 256
_ZR, _ZC = 16384, 1024
_BX = 2048
_BZ = 128
_GRID = _XR // _BX


def _copy_patch_kernel(x_ref, y_ref, z_ref, w_ref, xo_ref, zo_ref):
    i = pl.program_id(0)
    xo_ref[...] = x_ref[...]
    zo_ref[...] = z_ref[...]

    @pl.when(i == 0)
    def _patch():
        xo_ref[pl.ds(10, 1), :] = y_ref[pl.ds(0, 1), :]
        xo_ref[pl.ds(2, 1), :] = y_ref[pl.ds(1, 1), :]
        xo_ref[pl.ds(1, 1), :] = jnp.full((1, _XC), 45.0, jnp.float32)
        zrows = z_ref[pl.ds(0, 2), :]
        row = jax.lax.broadcasted_iota(jnp.int32, (2, _ZC), 0)
        col = jax.lax.broadcasted_iota(jnp.int32, (2, _ZC), 1)
        add = (jnp.where((row == 1) & (col == 3), w_ref[0], 0.0)
               + jnp.where((row == 0) & (col == 2), w_ref[1], 0.0)
               + jnp.where((row == 0) & (col == 1), w_ref[2], 0.0))
        zo_ref[pl.ds(0, 2), :] = zrows + add


def kernel(x, y, z, w):
    xo, zo = pl.pallas_call(
        _copy_patch_kernel,
        grid=(_GRID,),
        in_specs=[
            pl.BlockSpec((_BX, _XC), lambda i: (i, 0)),
            pl.BlockSpec((2, _XC), lambda i: (0, 0)),
            pl.BlockSpec((_BZ, _ZC), lambda i: (i, 0)),
            pl.BlockSpec(memory_space=pltpu.SMEM),
        ],
        out_specs=[
            pl.BlockSpec((_BX, _XC), lambda i: (i, 0)),
            pl.BlockSpec((_BZ, _ZC), lambda i: (i, 0)),
        ],
        out_shape=[
            jax.ShapeDtypeStruct((_XR, _XC), jnp.float32),
            jax.ShapeDtypeStruct((_ZR, _ZC), jnp.float32),
        ],
    )(x, y, z, w)
    return (xo, zo)


# speedup vs baseline: 1.0558x; 1.0558x over previous
import jax
import jax.numpy as jnp
from jax.experimental import pallas as pl
from jax.experimental.pallas import tpu as pltpu

_XR, _XC = 262144, 256
_ZR, _ZC = 16384, 1024
_BX = 8192
_BZ = 512
_GRID = _XR // _BX


def _copy_patch_kernel(x_ref, y_ref, z_ref, w_ref, xo_ref, zo_ref):
    i = pl.program_id(0)
    xo_ref[...] = x_ref[...]
    zo_ref[...] = z_ref[...]

    @pl.when(i == 0)
    def _patch():
        xo_ref[pl.ds(10, 1), :] = y_ref[pl.ds(0, 1), :]
        xo_ref[pl.ds(2, 1), :] = y_ref[pl.ds(1, 1), :]
        xo_ref[pl.ds(1, 1), :] = jnp.full((1, _XC), 45.0, jnp.float32)
        zrows = z_ref[pl.ds(0, 2), :]
        row = jax.lax.broadcasted_iota(jnp.int32, (2, _ZC), 0)
        col = jax.lax.broadcasted_iota(jnp.int32, (2, _ZC), 1)
        add = (jnp.where((row == 1) & (col == 3), w_ref[0], 0.0)
               + jnp.where((row == 0) & (col == 2), w_ref[1], 0.0)
               + jnp.where((row == 0) & (col == 1), w_ref[2], 0.0))
        zo_ref[pl.ds(0, 2), :] = zrows + add


def kernel(x, y, z, w):
    xo, zo = pl.pallas_call(
        _copy_patch_kernel,
        grid=(_GRID,),
        in_specs=[
            pl.BlockSpec((_BX, _XC), lambda i: (i, 0)),
            pl.BlockSpec((2, _XC), lambda i: (0, 0)),
            pl.BlockSpec((_BZ, _ZC), lambda i: (i, 0)),
            pl.BlockSpec(memory_space=pltpu.SMEM),
        ],
        out_specs=[
            pl.BlockSpec((_BX, _XC), lambda i: (i, 0)),
            pl.BlockSpec((_BZ, _ZC), lambda i: (i, 0)),
        ],
        out_shape=[
            jax.ShapeDtypeStruct((_XR, _XC), jnp.float32),
            jax.ShapeDtypeStruct((_ZR, _ZC), jnp.float32),
        ],
    )(x, y, z, w)
    return (xo, zo)
